# baseline (speedup 1.0000x reference)
import functools

import jax
import jax.numpy as jnp
from jax import lax
from jax.experimental import pallas as pl
from jax.experimental.pallas import tpu as pltpu
from jax.experimental.pallas import tpu_sc as plsc

_N = 10000
_E = 320000
_INC = 128
_OUTC = 128
_KV = 27

_NC = 2
_NS = 16
_NW = _NC * _NS
_C = 128
_CH = 80
_EPW = _C * _CH
_E_PAD = _NW * _EPW
_RPT = 632
_R = _NS * _RPT

_TN = 2000


def _zmm_body(z_ref, w_ref, out_ref):
    out_ref[0] = jnp.dot(z_ref[...], w_ref[0], preferred_element_type=jnp.float32)


def _final_body(p_ref, z_ref, wpt_ref, b_ref, out_ref):
    pt = jnp.dot(z_ref[...], wpt_ref[...], preferred_element_type=jnp.float32)
    out_ref[...] = p_ref[0] + p_ref[1] + pt + b_ref[0]


def _sc_body(zt_hbm, gidx_hbm, dst_hbm, out_hbm, ig, idb, rows, acc,
             sem, is1, is2):
    c = lax.axis_index("c")
    s = lax.axis_index("s")
    wid = c * _NS + s

    def _zr(i, carry):
        r = i // (_OUTC // 16)
        col = (i % (_OUTC // 16)) * 16
        rows[r, pl.ds(col, 16)] = jnp.zeros((16,), jnp.float32)
        return carry

    lax.fori_loop(0, _C * _OUTC // 16, _zr, 0)
    row0 = s * _RPT
    for t in range(0, _RPT, _C):
        sz = min(_C, _RPT - t)
        pltpu.sync_copy(rows.at[pl.ds(0, sz)], acc.at[pl.ds(row0 + t, sz)])
    plsc.subcore_barrier()

    base = wid * _EPW

    def _step(g, carry):
        off = base + g * _C
        pltpu.sync_copy(gidx_hbm.at[pl.ds(off, _C)], ig)
        pltpu.sync_copy(dst_hbm.at[pl.ds(off, _C)], idb)
        pltpu.async_copy(zt_hbm.at[ig], rows, sem).wait()
        pltpu.sync_copy(rows, acc.at[idb], add=True)
        return carry

    lax.fori_loop(0, _CH, _step, 0)
    plsc.subcore_barrier()
    pltpu.sync_copy(acc.at[pl.ds(row0, _RPT)], out_hbm.at[c, pl.ds(row0, _RPT)])


_sc_scatter = functools.partial(
    pl.kernel,
    out_type=jax.ShapeDtypeStruct((_NC, _R, _OUTC), jnp.float32),
    mesh=plsc.VectorSubcoreMesh(
        core_axis_name="c", subcore_axis_name="s",
        num_cores=_NC, num_subcores=_NS),
    scratch_types=[
        pltpu.VMEM((_C,), jnp.int32),
        pltpu.VMEM((_C,), jnp.int32),
        pltpu.VMEM((_C, _OUTC), jnp.float32),
        pltpu.VMEM_SHARED((_R, _OUTC), jnp.float32),
        pltpu.SemaphoreType.DMA,
        pltpu.SemaphoreType.DMA,
        pltpu.SemaphoreType.DMA,
    ],
)(_sc_body)


@jax.jit
def kernel(z_F, edge_index, kidx, W, W_pt, b_pt):
    src = edge_index[0]
    dst = edge_index[1]

    Z = pl.pallas_call(
        _zmm_body,
        grid=(_N // _TN, _KV),
        in_specs=[
            pl.BlockSpec((_TN, _INC), lambda n, k: (n, 0)),
            pl.BlockSpec((1, _INC, _OUTC), lambda n, k: (k, 0, 0)),
        ],
        out_specs=pl.BlockSpec((1, _TN, _OUTC), lambda n, k: (k, n, 0)),
        out_shape=jax.ShapeDtypeStruct((_KV, _N, _OUTC), jnp.float32),
    )(z_F, W)
    Zt = Z.reshape(_KV * _N, _OUTC)

    gidx = kidx * _N + src
    pad = _E_PAD - _E
    gidx = jnp.concatenate([gidx, jnp.zeros((pad,), jnp.int32)])
    dstp = jnp.concatenate([dst, jnp.full((pad,), _N, jnp.int32)])

    partials = _sc_scatter(Zt, gidx, dstp)

    out = pl.pallas_call(
        _final_body,
        grid=(_N // _TN,),
        in_specs=[
            pl.BlockSpec((_NC, _TN, _OUTC), lambda n: (0, n, 0)),
            pl.BlockSpec((_TN, _INC), lambda n: (n, 0)),
            pl.BlockSpec((_INC, _OUTC), lambda n: (0, 0)),
            pl.BlockSpec((1, _OUTC), lambda n: (0, 0)),
        ],
        out_specs=pl.BlockSpec((_TN, _OUTC), lambda n: (n, 0)),
        out_shape=jax.ShapeDtypeStruct((_N, _OUTC), jnp.float32),
    )(partials, z_F, W_pt, b_pt.reshape(1, _OUTC))
    return out


# speedup vs baseline: 1.2344x; 1.2344x over previous
import functools

import jax
import jax.numpy as jnp
from jax import lax
from jax.experimental import pallas as pl
from jax.experimental.pallas import tpu as pltpu
from jax.experimental.pallas import tpu_sc as plsc

_N = 10000
_E = 320000
_INC = 128
_OUTC = 128
_KV = 27

_NC = 2
_NS = 16
_NW = _NC * _NS
_C = 128
_CH = 79
_EPW = _C * _CH
_E_PAD = _NW * _EPW
_RPT = 632
_R = _NS * _RPT

_TN = 2000


def _zmm_body(z_ref, w_ref, out_ref):
    out_ref[0] = jnp.dot(z_ref[...], w_ref[0], preferred_element_type=jnp.float32)


def _final_body(p_ref, z_ref, wpt_ref, b_ref, out_ref):
    pt = jnp.dot(z_ref[...], wpt_ref[...], preferred_element_type=jnp.float32)
    out_ref[...] = p_ref[0] + p_ref[1] + pt + b_ref[0]


def _sc_body(zt_hbm, gidx_hbm, dst_hbm, out_hbm, ig, idb, rows, acc, sem):
    c = lax.axis_index("c")
    s = lax.axis_index("s")
    wid = c * _NS + s

    def _zr(i, carry):
        r = i // (_OUTC // 16)
        col = (i % (_OUTC // 16)) * 16
        rows[r, pl.ds(col, 16)] = jnp.zeros((16,), jnp.float32)
        return carry

    lax.fori_loop(0, _C * _OUTC // 16, _zr, 0)
    row0 = s * _RPT
    for t in range(0, _RPT, _C):
        sz = min(_C, _RPT - t)
        pltpu.sync_copy(rows.at[pl.ds(0, sz)], acc.at[pl.ds(row0 + t, sz)])
    plsc.subcore_barrier()

    base = wid * _EPW

    def _step(g, carry):
        off = base + g * _C
        pltpu.sync_copy(gidx_hbm.at[pl.ds(off, _C)], ig)
        pltpu.sync_copy(dst_hbm.at[pl.ds(off, _C)], idb)
        pltpu.async_copy(zt_hbm.at[ig], rows, sem).wait()
        pltpu.sync_copy(rows, acc.at[idb], add=True)
        return carry

    lax.fori_loop(0, _CH, _step, 0)
    plsc.subcore_barrier()
    pltpu.sync_copy(acc.at[pl.ds(row0, _RPT)], out_hbm.at[c, pl.ds(row0, _RPT)])


_sc_scatter = functools.partial(
    pl.kernel,
    out_type=jax.ShapeDtypeStruct((_NC, _R, _OUTC), jnp.float32),
    mesh=plsc.VectorSubcoreMesh(
        core_axis_name="c", subcore_axis_name="s",
        num_cores=_NC, num_subcores=_NS),
    scratch_types=[
        pltpu.VMEM((_C,), jnp.int32),
        pltpu.VMEM((_C,), jnp.int32),
        pltpu.VMEM((_C, _OUTC), jnp.float32),
        pltpu.VMEM_SHARED((_R, _OUTC), jnp.float32),
        pltpu.SemaphoreType.DMA,
    ],
)(_sc_body)


@jax.jit
def kernel(z_F, edge_index, kidx, W, W_pt, b_pt):
    src = edge_index[0]
    dst = edge_index[1]

    Z = pl.pallas_call(
        _zmm_body,
        grid=(_N // _TN, _KV),
        in_specs=[
            pl.BlockSpec((_TN, _INC), lambda n, k: (n, 0)),
            pl.BlockSpec((1, _INC, _OUTC), lambda n, k: (k, 0, 0)),
        ],
        out_specs=pl.BlockSpec((1, _TN, _OUTC), lambda n, k: (k, n, 0)),
        out_shape=jax.ShapeDtypeStruct((_KV, _N, _OUTC), jnp.float32),
    )(z_F, W)
    Zt = Z.reshape(_KV * _N, _OUTC)

    gidx = kidx * _N + src
    pad = _E_PAD - _E
    gidx = jnp.concatenate([gidx, jnp.zeros((pad,), jnp.int32)])
    dstp = jnp.concatenate([dst, jnp.full((pad,), _N, jnp.int32)])

    partials = _sc_scatter(Zt, gidx, dstp)

    out = pl.pallas_call(
        _final_body,
        grid=(_N // _TN,),
        in_specs=[
            pl.BlockSpec((_NC, _TN, _OUTC), lambda n: (0, n, 0)),
            pl.BlockSpec((_TN, _INC), lambda n: (n, 0)),
            pl.BlockSpec((_INC, _OUTC), lambda n: (0, 0)),
            pl.BlockSpec((1, _OUTC), lambda n: (0, 0)),
        ],
        out_specs=pl.BlockSpec((_TN, _OUTC), lambda n: (n, 0)),
        out_shape=jax.ShapeDtypeStruct((_N, _OUTC), jnp.float32),
    )(partials, z_F, W_pt, b_pt.reshape(1, _OUTC))
    return out
